# baseline (speedup 1.0000x reference)
import jax
import jax.numpy as jnp
from jax import lax
from jax.experimental import pallas as pl
from jax.experimental.pallas import tpu as pltpu


def _make_body(B, T):
    def body(t_ref, o_ref, sem):
        def fire(i, c):
            b = i // T
            t = i - b * T
            pltpu.make_async_copy(t_ref, o_ref.at[b, t], sem).start()
            return c

        lax.fori_loop(0, B * T, fire, 0)

        def drain(i, c):
            b = i // T
            t = i - b * T
            pltpu.make_async_copy(t_ref, o_ref.at[b, t], sem).wait()
            return c

        lax.fori_loop(0, B * T, drain, 0)

    return body


def kernel(x, table):
    B, T, N, _ = x.shape
    D = table.shape[1]
    rows = N * D // 128
    t2 = table.reshape(rows, 128)
    out = pl.pallas_call(
        _make_body(B, T),
        in_specs=[pl.BlockSpec(memory_space=pltpu.VMEM)],
        out_specs=pl.BlockSpec(memory_space=pl.ANY),
        out_shape=jax.ShapeDtypeStruct((B, T, rows, 128), jnp.float32),
        scratch_shapes=[pltpu.SemaphoreType.DMA],
    )(t2)
    return out.reshape(B, T, N, D)


# speedup vs baseline: 17.5371x; 17.5371x over previous
import jax
import jax.numpy as jnp
from jax import lax
from jax.experimental import pallas as pl
from jax.experimental.pallas import tpu as pltpu


def _make_body(B, T):
    def body(t_ref, o_ref, sem):
        def fire(i, c):
            b = i // T
            t = i - b * T
            pltpu.make_async_copy(t_ref, o_ref.at[b, t], sem).start()
            return c

        lax.fori_loop(0, B * T, fire, 0)

        def drain(i, c):
            b = i // T
            t = i - b * T
            pltpu.make_async_copy(t_ref, o_ref.at[b, t], sem).wait()
            return c

        lax.fori_loop(0, B * T, drain, 0)

    return body


def kernel(x, table):
    B, T, N, _ = x.shape
    D = table.shape[1]
    t2 = table.T
    out = pl.pallas_call(
        _make_body(B, T),
        in_specs=[pl.BlockSpec(memory_space=pltpu.VMEM)],
        out_specs=pl.BlockSpec(memory_space=pl.ANY),
        out_shape=jax.ShapeDtypeStruct((B, T, D, N), jnp.float32),
        scratch_shapes=[pltpu.SemaphoreType.DMA],
    )(t2)
    return out.transpose(0, 1, 3, 2)
